# baseline (speedup 1.0000x reference)
import functools

import jax
import jax.numpy as jnp
from jax import lax
from jax.experimental import pallas as pl
from jax.experimental.pallas import tpu as pltpu

B, L, D = 2, 2048, 768
E, TOPK, H = 8, 2, 3072
L_HEAD = L // 3
L_WRIST = L // 3
P_START = L_HEAD + L_WRIST
N_HP = L_HEAD + (L - P_START)

MB = 512
HB = H // 2
NM = L // MB
NH = H // HB


def _gate_body(x_ref, tc_ref, gw_ref, gb_ref, tw_ref, tb_ref,
               slab_ref, c_ref):
    tok = lax.broadcasted_iota(jnp.int32, (L, 1), 0)
    hm = (tok < L_HEAD).astype(jnp.float32)
    pm = (tok >= P_START).astype(jnp.float32)
    wm = 1.0 - hm - pm
    hp_mask = hm + pm
    wp_mask = wm + pm
    ones_col = jnp.ones((L, 1), jnp.float32)

    slab_ref[6] = 0
    slab_ref[7] = 0
    for b in range(B):
        xb = x_ref[b]
        hs = jnp.sum(xb * hm, axis=0, keepdims=True)
        ps = jnp.sum(xb * pm, axis=0, keepdims=True)
        ts = jnp.sum(xb, axis=0, keepdims=True)
        ws = ts - hs - ps
        full = ts * (1.0 / L)
        hp = (hs + ps) * (1.0 / N_HP)
        wp = (ws + ps) * (1.0 / N_HP)
        gi = jnp.concatenate([full, hp, wp], axis=1)
        logits = lax.dot_general(
            gi, gw_ref[...], (((1,), (0,)), ((), ())),
            precision=lax.Precision.HIGHEST,
            preferred_element_type=jnp.float32) + gb_ref[...]
        tcb = tc_ref[b:b + 1]
        mod = lax.dot_general(
            jax.nn.silu(tcb), tw_ref[...], (((1,), (0,)), ((), ())),
            precision=lax.Precision.HIGHEST,
            preferred_element_type=jnp.float32) + tb_ref[...]
        scale = mod[:, :E]
        shift = mod[:, E:]
        logits = logits * (1.0 + scale) + shift
        z = logits - jnp.max(logits)
        ez = jnp.exp(z)
        s = ez / jnp.sum(ez)
        lane = lax.broadcasted_iota(jnp.int32, (1, E), 1)
        m1 = jnp.max(s)
        i1 = jnp.min(jnp.where(s == m1, lane, E))
        s2 = jnp.where(lane == i1, -1.0, s)
        m2 = jnp.max(s2)
        i2 = jnp.min(jnp.where(s2 == m2, lane, E))
        den = m1 + m2 + 1e-8
        w1 = m1 / den
        w2 = m2 / den
        slab_ref[3 * b + 0] = 0
        slab_ref[3 * b + 1] = i1
        slab_ref[3 * b + 2] = i2
        c_ref[3 * b + 0] = ones_col
        for slot, iv, wv in ((1, i1, w1), (2, i2, w2)):
            mask = jnp.where(iv == 1, hp_mask,
                             jnp.where(iv == 2, wp_mask, ones_col))
            c_ref[3 * b + slot] = wv * mask


def _moe_body(s_ref, x_ref, ew1_ref, ew2_ref, sw1_ref, sw2_ref,
              eb1_ref, eb2_ref, sb1_ref, sb2_ref, c_ref,
              o_ref, acc_ref, w1s_ref, w2s_ref):
    j = pl.program_id(1)
    hb = pl.program_id(2)
    m = pl.program_id(3)
    x = x_ref[0].astype(jnp.bfloat16)
    c = c_ref[0]
    rows = pl.ds(m * MB, MB)

    @pl.when(m == 0)
    def _():
        @pl.when(j == 0)
        def _():
            w1s_ref[...] = sw1_ref[...].astype(jnp.bfloat16)
            w2s_ref[...] = sw2_ref[...].astype(jnp.bfloat16)

        @pl.when(j > 0)
        def _():
            w1s_ref[...] = ew1_ref[0].astype(jnp.bfloat16)
            w2s_ref[...] = ew2_ref[0].astype(jnp.bfloat16)

    def compute(b1, b2):
        h = lax.dot_general(x, w1s_ref[...], (((1,), (1,)), ((), ())),
                            preferred_element_type=jnp.float32) + b1
        h = jax.nn.gelu(h, approximate=True).astype(jnp.bfloat16)
        y = lax.dot_general(h, w2s_ref[...], (((1,), (1,)), ((), ())),
                            preferred_element_type=jnp.float32)
        y = jnp.where(hb == 0, y + b2, y)
        return c * y

    @pl.when(jnp.logical_and(j == 0, hb == 0))
    def _():
        acc_ref[rows, :] = compute(sb1_ref[...], sb2_ref[...])

    @pl.when(jnp.logical_and(j == 0, hb == 1))
    def _():
        acc_ref[rows, :] += compute(sb1_ref[...], sb2_ref[...])

    active = jnp.max(jnp.abs(c)) > 0.0

    @pl.when(jnp.logical_and(j > 0, active))
    def _():
        acc_ref[rows, :] += compute(eb1_ref[0], eb2_ref[0])

    @pl.when(jnp.logical_and(j == 2, hb == NH - 1))
    def _():
        o_ref[0] = acc_ref[rows, :]


def _sel(b, j, s_ref):
    return s_ref[3 * b + jnp.maximum(j, 1)]


@jax.jit
def kernel(context_c, time_cond, gate_W, gate_b, tmod_W, tmod_b,
           eW1, eb1, eW2, eb2, sW1, sb1, sW2, sb2):
    slab, coeff = pl.pallas_call(
        _gate_body,
        out_shape=[
            jax.ShapeDtypeStruct((8,), jnp.int32),
            jax.ShapeDtypeStruct((3 * B, L, 1), jnp.float32),
        ],
        out_specs=[
            pl.BlockSpec(memory_space=pltpu.SMEM),
            pl.BlockSpec(memory_space=pltpu.VMEM),
        ],
    )(context_c, time_cond, gate_W.T, gate_b.reshape(1, E),
      tmod_W.T, tmod_b.reshape(1, 2 * E))

    out = pl.pallas_call(
        _moe_body,
        grid_spec=pltpu.PrefetchScalarGridSpec(
            num_scalar_prefetch=1,
            grid=(B, 3, NH, NM),
            in_specs=[
                pl.BlockSpec((1, MB, D), lambda b, j, hb, m, s: (b, m, 0)),
                pl.BlockSpec((1, HB, D),
                             lambda b, j, hb, m, s: (
                                 _sel(b, j, s),
                                 jnp.where(j == 0, 0, hb), 0)),
                pl.BlockSpec((1, D, HB),
                             lambda b, j, hb, m, s: (
                                 _sel(b, j, s), 0,
                                 jnp.where(j == 0, 0, hb))),
                pl.BlockSpec((HB, D),
                             lambda b, j, hb, m, s: (
                                 jnp.where(j == 0, hb, NH - 1), 0)),
                pl.BlockSpec((D, HB),
                             lambda b, j, hb, m, s: (
                                 0, jnp.where(j == 0, hb, NH - 1))),
                pl.BlockSpec((1, 1, HB),
                             lambda b, j, hb, m, s: (
                                 _sel(b, j, s), 0,
                                 jnp.where(j == 0, 0, hb))),
                pl.BlockSpec((1, 1, D),
                             lambda b, j, hb, m, s: (_sel(b, j, s), 0, 0)),
                pl.BlockSpec((1, HB),
                             lambda b, j, hb, m, s: (
                                 0, jnp.where(j == 0, hb, NH - 1))),
                pl.BlockSpec((1, D), lambda b, j, hb, m, s: (0, 0)),
                pl.BlockSpec((1, MB, 1),
                             lambda b, j, hb, m, s: (3 * b + j, m, 0)),
            ],
            out_specs=pl.BlockSpec(
                (1, MB, D),
                lambda b, j, hb, m, s: (
                    b,
                    jnp.where(jnp.logical_and(j == 2, hb == NH - 1), m, 0),
                    0)),
            scratch_shapes=[pltpu.VMEM((L, D), jnp.float32),
                            pltpu.VMEM((HB, D), jnp.bfloat16),
                            pltpu.VMEM((D, HB), jnp.bfloat16)],
        ),
        out_shape=jax.ShapeDtypeStruct((B, L, D), jnp.float32),
    )(slab, context_c, eW1, eW2, sW1, sW2,
      eb1.reshape(E, 1, H), eb2.reshape(E, 1, D),
      sb1.reshape(1, H), sb2.reshape(1, D), coeff)
    return out


# speedup vs baseline: 1.0089x; 1.0089x over previous
import functools

import jax
import jax.numpy as jnp
from jax import lax
from jax.experimental import pallas as pl
from jax.experimental.pallas import tpu as pltpu

B, L, D = 2, 2048, 768
E, TOPK, H = 8, 2, 3072
L_HEAD = L // 3
L_WRIST = L // 3
P_START = L_HEAD + L_WRIST
N_HP = L_HEAD + (L - P_START)

MB = 1024
HB = H // 4
NM = L // MB
NH = H // HB


def _gate_body(x_ref, tc_ref, gw_ref, gb_ref, tw_ref, tb_ref,
               slab_ref, c_ref, x16_ref):
    x16_ref[...] = x_ref[...].astype(jnp.bfloat16)
    tok = lax.broadcasted_iota(jnp.int32, (L, 1), 0)
    hm = (tok < L_HEAD).astype(jnp.float32)
    pm = (tok >= P_START).astype(jnp.float32)
    wm = 1.0 - hm - pm
    hp_mask = hm + pm
    wp_mask = wm + pm
    ones_col = jnp.ones((L, 1), jnp.float32)

    slab_ref[6] = 0
    slab_ref[7] = 0
    for b in range(B):
        xb = x_ref[b]
        hs = jnp.sum(xb * hm, axis=0, keepdims=True)
        ps = jnp.sum(xb * pm, axis=0, keepdims=True)
        ts = jnp.sum(xb, axis=0, keepdims=True)
        ws = ts - hs - ps
        full = ts * (1.0 / L)
        hp = (hs + ps) * (1.0 / N_HP)
        wp = (ws + ps) * (1.0 / N_HP)
        gi = jnp.concatenate([full, hp, wp], axis=1)
        logits = lax.dot_general(
            gi, gw_ref[...], (((1,), (0,)), ((), ())),
            precision=lax.Precision.HIGHEST,
            preferred_element_type=jnp.float32) + gb_ref[...]
        tcb = tc_ref[b:b + 1]
        mod = lax.dot_general(
            jax.nn.silu(tcb), tw_ref[...], (((1,), (0,)), ((), ())),
            precision=lax.Precision.HIGHEST,
            preferred_element_type=jnp.float32) + tb_ref[...]
        scale = mod[:, :E]
        shift = mod[:, E:]
        logits = logits * (1.0 + scale) + shift
        z = logits - jnp.max(logits)
        ez = jnp.exp(z)
        s = ez / jnp.sum(ez)
        lane = lax.broadcasted_iota(jnp.int32, (1, E), 1)
        m1 = jnp.max(s)
        i1 = jnp.min(jnp.where(s == m1, lane, E))
        s2 = jnp.where(lane == i1, -1.0, s)
        m2 = jnp.max(s2)
        i2 = jnp.min(jnp.where(s2 == m2, lane, E))
        den = m1 + m2 + 1e-8
        w1 = m1 / den
        w2 = m2 / den
        slab_ref[3 * b + 0] = 0
        slab_ref[3 * b + 1] = i1
        slab_ref[3 * b + 2] = i2
        c_ref[3 * b + 0] = ones_col
        for slot, iv, wv in ((1, i1, w1), (2, i2, w2)):
            mask = jnp.where(iv == 1, hp_mask,
                             jnp.where(iv == 2, wp_mask, ones_col))
            c_ref[3 * b + slot] = wv * mask


def _moe_body(s_ref, x_ref, ew1_ref, ew2_ref, sw1_ref, sw2_ref,
              eb1_ref, eb2_ref, sb1_ref, sb2_ref, c_ref,
              o_ref, acc_ref, w1s_ref, w2s_ref):
    j = pl.program_id(1)
    hb = pl.program_id(2)
    m = pl.program_id(3)
    x = x_ref[0]
    c = c_ref[0]
    rows = pl.ds(m * MB, MB)

    @pl.when(m == 0)
    def _():
        @pl.when(j == 0)
        def _():
            w1s_ref[...] = sw1_ref[...].astype(jnp.bfloat16)
            w2s_ref[...] = sw2_ref[...].astype(jnp.bfloat16)

        @pl.when(j > 0)
        def _():
            w1s_ref[...] = ew1_ref[0].astype(jnp.bfloat16)
            w2s_ref[...] = ew2_ref[0].astype(jnp.bfloat16)

    def compute(b1, b2, add_b2):
        h = lax.dot_general(x, w1s_ref[...], (((1,), (1,)), ((), ())),
                            preferred_element_type=jnp.float32) + b1
        h = jax.nn.gelu(h.astype(jnp.bfloat16), approximate=True)
        y = lax.dot_general(h, w2s_ref[...], (((1,), (1,)), ((), ())),
                            preferred_element_type=jnp.float32)
        if add_b2:
            y = y + b2
        return c * y

    @pl.when(jnp.logical_and(j == 0, hb == 0))
    def _():
        acc_ref[rows, :] = compute(sb1_ref[...], sb2_ref[...], True)

    @pl.when(jnp.logical_and(j == 0, hb > 0))
    def _():
        acc_ref[rows, :] += compute(sb1_ref[...], sb2_ref[...], False)

    active = jnp.max(jnp.abs(c)) > 0.0

    @pl.when(jnp.logical_and(j > 0, jnp.logical_and(hb == 0, active)))
    def _():
        acc_ref[rows, :] += compute(eb1_ref[0], eb2_ref[0], True)

    @pl.when(jnp.logical_and(j > 0, jnp.logical_and(hb > 0, active)))
    def _():
        acc_ref[rows, :] += compute(eb1_ref[0], eb2_ref[0], False)

    @pl.when(jnp.logical_and(j == 2, hb == NH - 1))
    def _():
        o_ref[0] = acc_ref[rows, :]


def _sel(b, j, s_ref):
    return s_ref[3 * b + jnp.maximum(j, 1)]


@jax.jit
def kernel(context_c, time_cond, gate_W, gate_b, tmod_W, tmod_b,
           eW1, eb1, eW2, eb2, sW1, sb1, sW2, sb2):
    slab, coeff, x16 = pl.pallas_call(
        _gate_body,
        out_shape=[
            jax.ShapeDtypeStruct((8,), jnp.int32),
            jax.ShapeDtypeStruct((3 * B, L, 1), jnp.float32),
            jax.ShapeDtypeStruct((B, L, D), jnp.bfloat16),
        ],
        out_specs=[
            pl.BlockSpec(memory_space=pltpu.SMEM),
            pl.BlockSpec(memory_space=pltpu.VMEM),
            pl.BlockSpec(memory_space=pltpu.VMEM),
        ],
    )(context_c, time_cond, gate_W.T, gate_b.reshape(1, E),
      tmod_W.T, tmod_b.reshape(1, 2 * E))

    out = pl.pallas_call(
        _moe_body,
        grid_spec=pltpu.PrefetchScalarGridSpec(
            num_scalar_prefetch=1,
            grid=(B, 3, NH, NM),
            in_specs=[
                pl.BlockSpec((1, MB, D), lambda b, j, hb, m, s: (b, m, 0)),
                pl.BlockSpec((1, HB, D),
                             lambda b, j, hb, m, s: (
                                 _sel(b, j, s),
                                 jnp.where(j == 0, 0, hb), 0)),
                pl.BlockSpec((1, D, HB),
                             lambda b, j, hb, m, s: (
                                 _sel(b, j, s), 0,
                                 jnp.where(j == 0, 0, hb))),
                pl.BlockSpec((HB, D),
                             lambda b, j, hb, m, s: (
                                 jnp.where(j == 0, hb, NH - 1), 0)),
                pl.BlockSpec((D, HB),
                             lambda b, j, hb, m, s: (
                                 0, jnp.where(j == 0, hb, NH - 1))),
                pl.BlockSpec((1, 1, HB),
                             lambda b, j, hb, m, s: (
                                 _sel(b, j, s), 0,
                                 jnp.where(j == 0, 0, hb))),
                pl.BlockSpec((1, 1, D),
                             lambda b, j, hb, m, s: (_sel(b, j, s), 0, 0)),
                pl.BlockSpec((1, HB),
                             lambda b, j, hb, m, s: (
                                 0, jnp.where(j == 0, hb, NH - 1))),
                pl.BlockSpec((1, D), lambda b, j, hb, m, s: (0, 0)),
                pl.BlockSpec((1, MB, 1),
                             lambda b, j, hb, m, s: (3 * b + j, m, 0)),
            ],
            out_specs=pl.BlockSpec(
                (1, MB, D),
                lambda b, j, hb, m, s: (
                    b,
                    jnp.where(jnp.logical_and(j == 2, hb == NH - 1), m, 0),
                    0)),
            scratch_shapes=[pltpu.VMEM((L, D), jnp.float32),
                            pltpu.VMEM((HB, D), jnp.bfloat16),
                            pltpu.VMEM((D, HB), jnp.bfloat16)],
        ),
        out_shape=jax.ShapeDtypeStruct((B, L, D), jnp.float32),
    )(slab, x16, eW1, eW2, sW1, sW2,
      eb1.reshape(E, 1, H), eb2.reshape(E, 1, D),
      sb1.reshape(1, H), sb2.reshape(1, D), coeff)
    return out


# speedup vs baseline: 1.0575x; 1.0482x over previous
import functools

import jax
import jax.numpy as jnp
from jax import lax
from jax.experimental import pallas as pl
from jax.experimental.pallas import tpu as pltpu

B, L, D = 2, 2048, 768
E, TOPK, H = 8, 2, 3072
L_HEAD = L // 3
L_WRIST = L // 3
P_START = L_HEAD + L_WRIST
N_HP = L_HEAD + (L - P_START)

MB = 256
HB = 384
NM = L // MB
NH = H // HB
T = 3 * NH + NM


def _gate_body(x_ref, tc_ref, gw_ref, gb_ref, tw_ref, tb_ref,
               slab_ref, c_ref, x16_ref):
    x16_ref[...] = x_ref[...].astype(jnp.bfloat16)
    tok = lax.broadcasted_iota(jnp.int32, (L, 1), 0)
    hm = (tok < L_HEAD).astype(jnp.float32)
    pm = (tok >= P_START).astype(jnp.float32)
    wm = 1.0 - hm - pm
    hp_mask = hm + pm
    wp_mask = wm + pm
    ones_col = jnp.ones((L, 1), jnp.float32)

    slab_ref[6] = 0
    slab_ref[7] = 0
    for b in range(B):
        xb = x_ref[b]
        hs = jnp.sum(xb * hm, axis=0, keepdims=True)
        ps = jnp.sum(xb * pm, axis=0, keepdims=True)
        ts = jnp.sum(xb, axis=0, keepdims=True)
        ws = ts - hs - ps
        full = ts * (1.0 / L)
        hp = (hs + ps) * (1.0 / N_HP)
        wp = (ws + ps) * (1.0 / N_HP)
        gi = jnp.concatenate([full, hp, wp], axis=1)
        logits = lax.dot_general(
            gi, gw_ref[...], (((1,), (0,)), ((), ())),
            precision=lax.Precision.HIGHEST,
            preferred_element_type=jnp.float32) + gb_ref[...]
        tcb = tc_ref[b:b + 1]
        mod = lax.dot_general(
            jax.nn.silu(tcb), tw_ref[...], (((1,), (0,)), ((), ())),
            precision=lax.Precision.HIGHEST,
            preferred_element_type=jnp.float32) + tb_ref[...]
        scale = mod[:, :E]
        shift = mod[:, E:]
        logits = logits * (1.0 + scale) + shift
        z = logits - jnp.max(logits)
        ez = jnp.exp(z)
        s = ez / jnp.sum(ez)
        lane = lax.broadcasted_iota(jnp.int32, (1, E), 1)
        m1 = jnp.max(s)
        i1 = jnp.min(jnp.where(s == m1, lane, E))
        s2 = jnp.where(lane == i1, -1.0, s)
        m2 = jnp.max(s2)
        i2 = jnp.min(jnp.where(s2 == m2, lane, E))
        den = m1 + m2 + 1e-8
        w1 = m1 / den
        w2 = m2 / den
        slab_ref[3 * b + 0] = 0
        slab_ref[3 * b + 1] = i1
        slab_ref[3 * b + 2] = i2
        c_ref[3 * b + 0] = ones_col
        for slot, iv, wv in ((1, i1, w1), (2, i2, w2)):
            mask = jnp.where(iv == 1, hp_mask,
                             jnp.where(iv == 2, wp_mask, ones_col))
            c_ref[3 * b + slot] = wv * mask


def _moe_body(s_ref, x_ref, ew1_ref, ew2_ref, sw1_ref, sw2_ref,
              eb1_ref, eb2_ref, sb1_ref, sb2_ref, c1_ref, c2_ref,
              o_ref, w1c_ref, w2c_ref, b1c_ref, b2s_ref):
    b = pl.program_id(0)
    t = pl.program_id(1)
    tf = t - NH
    e_ord = jnp.clip(tf // NH, 0, 1)
    echunk = jnp.clip(tf, 0, 2 * NH - 1) % NH

    @pl.when(jnp.logical_and(b == 0, t < NH))
    def _():
        off = t * HB
        w1c_ref[pl.ds(off, HB), :] = sw1_ref[...].astype(jnp.bfloat16)
        w2c_ref[:, pl.ds(off, HB)] = sw2_ref[...].astype(jnp.bfloat16)
        b1c_ref[:, pl.ds(off, HB)] = sb1_ref[...]

    @pl.when(jnp.logical_and(t >= NH, t < 3 * NH))
    def _():
        off = (1 + e_ord) * H + echunk * HB
        w1c_ref[pl.ds(off, HB), :] = ew1_ref[0].astype(jnp.bfloat16)
        w2c_ref[:, pl.ds(off, HB)] = ew2_ref[0].astype(jnp.bfloat16)
        b1c_ref[:, pl.ds(off, HB)] = eb1_ref[0]

    @pl.when(t == NH)
    def _():
        b2s_ref[0:1, :] = eb2_ref[0]

    @pl.when(t == 2 * NH)
    def _():
        b2s_ref[1:2, :] = eb2_ref[0]

    @pl.when(t >= 3 * NH)
    def _():
        x = x_ref[0]
        c1 = c1_ref[0]
        c2 = c2_ref[0]
        h = lax.dot_general(x, w1c_ref[...], (((1,), (1,)), ((), ())),
                            preferred_element_type=jnp.float32)
        h = jax.nn.gelu((h + b1c_ref[...]).astype(jnp.bfloat16),
                        approximate=True)
        dn = (((1,), (1,)), ((), ()))
        y0 = lax.dot_general(h[:, :H], w2c_ref[:, :H], dn,
                             preferred_element_type=jnp.float32)
        y1 = lax.dot_general(h[:, H:2 * H], w2c_ref[:, H:2 * H], dn,
                             preferred_element_type=jnp.float32)
        y2 = lax.dot_general(h[:, 2 * H:], w2c_ref[:, 2 * H:], dn,
                             preferred_element_type=jnp.float32)
        o_ref[0] = (y0 + sb2_ref[...]
                    + c1 * (y1 + b2s_ref[0:1, :])
                    + c2 * (y2 + b2s_ref[1:2, :]))


def _mclip(t):
    return jnp.clip(t - 3 * NH, 0, NM - 1)


def _eidx(b, t, s_ref):
    tf = t - NH
    e_ord = jnp.clip(tf // NH, 0, 1)
    echunk = jnp.clip(tf, 0, 2 * NH - 1) % NH
    return s_ref[3 * b + 1 + e_ord], echunk


def _schunk(b, t):
    return jnp.where(b == 0, jnp.clip(t, 0, NH - 1), NH - 1)


@jax.jit
def kernel(context_c, time_cond, gate_W, gate_b, tmod_W, tmod_b,
           eW1, eb1, eW2, eb2, sW1, sb1, sW2, sb2):
    slab, coeff, x16 = pl.pallas_call(
        _gate_body,
        out_shape=[
            jax.ShapeDtypeStruct((8,), jnp.int32),
            jax.ShapeDtypeStruct((3 * B, L, 1), jnp.float32),
            jax.ShapeDtypeStruct((B, L, D), jnp.bfloat16),
        ],
        out_specs=[
            pl.BlockSpec(memory_space=pltpu.SMEM),
            pl.BlockSpec(memory_space=pltpu.VMEM),
            pl.BlockSpec(memory_space=pltpu.VMEM),
        ],
    )(context_c, time_cond, gate_W.T, gate_b.reshape(1, E),
      tmod_W.T, tmod_b.reshape(1, 2 * E))

    out = pl.pallas_call(
        _moe_body,
        grid_spec=pltpu.PrefetchScalarGridSpec(
            num_scalar_prefetch=1,
            grid=(B, T),
            in_specs=[
                pl.BlockSpec((1, MB, D), lambda b, t, s: (b, _mclip(t), 0)),
                pl.BlockSpec((1, HB, D),
                             lambda b, t, s: (
                                 _eidx(b, t, s)[0], _eidx(b, t, s)[1], 0)),
                pl.BlockSpec((1, D, HB),
                             lambda b, t, s: (
                                 _eidx(b, t, s)[0], 0, _eidx(b, t, s)[1])),
                pl.BlockSpec((HB, D), lambda b, t, s: (_schunk(b, t), 0)),
                pl.BlockSpec((D, HB), lambda b, t, s: (0, _schunk(b, t))),
                pl.BlockSpec((1, 1, HB),
                             lambda b, t, s: (
                                 _eidx(b, t, s)[0], 0, _eidx(b, t, s)[1])),
                pl.BlockSpec((1, 1, D),
                             lambda b, t, s: (_eidx(b, t, s)[0], 0, 0)),
                pl.BlockSpec((1, HB), lambda b, t, s: (0, _schunk(b, t))),
                pl.BlockSpec((1, D), lambda b, t, s: (0, 0)),
                pl.BlockSpec((1, MB, 1),
                             lambda b, t, s: (3 * b + 1, _mclip(t), 0)),
                pl.BlockSpec((1, MB, 1),
                             lambda b, t, s: (3 * b + 2, _mclip(t), 0)),
            ],
            out_specs=pl.BlockSpec(
                (1, MB, D), lambda b, t, s: (b, _mclip(t), 0)),
            scratch_shapes=[pltpu.VMEM((3 * H, D), jnp.bfloat16),
                            pltpu.VMEM((D, 3 * H), jnp.bfloat16),
                            pltpu.VMEM((1, 3 * H), jnp.float32),
                            pltpu.VMEM((8, D), jnp.float32)],
        ),
        out_shape=jax.ShapeDtypeStruct((B, L, D), jnp.float32),
    )(slab, x16, eW1, eW2, sW1, sW2,
      eb1.reshape(E, 1, H), eb2.reshape(E, 1, D),
      sb1.reshape(1, H), sb2.reshape(1, D), coeff, coeff)
    return out
